# Initial kernel scaffold; baseline (speedup 1.0000x reference)
import jax
import jax.numpy as jnp
from jax.experimental import pallas as pl


def kernel(feat, edge_index, W, b, gamma, beta):
    raise NotImplementedError("write your pallas kernel here")


# speedup vs baseline: 3.0027x; 3.0027x over previous
import functools

import jax
import jax.numpy as jnp
from jax import lax
from jax.experimental import pallas as pl
from jax.experimental.pallas import tpu as pltpu
from jax.experimental.pallas import tpu_sc as plsc

N = 10000
NP = 10240
E = 160000
D = 256
HALF = 128
NC, NS, LANES = 2, 16, 16
ET = E // NS
K = 80
NB = ET // K
RPT = NP // NS
BN_EPS = 1e-5
BR = 2048
GRID = NP // BR

_mesh = plsc.VectorSubcoreMesh(
    core_axis_name="c", subcore_axis_name="s", num_cores=NC, num_subcores=NS
)


def _fill1d(ref, n, val):
    def body(i, carry):
        ref[pl.ds(i * LANES, LANES)] = jnp.full((LANES,), val, jnp.float32)
        return carry
    lax.fori_loop(0, n // LANES, body, 0)


@functools.partial(
    pl.kernel,
    out_type=jax.ShapeDtypeStruct((2 * NP,), jnp.float32),
    mesh=_mesh,
    scratch_types=[
        pltpu.VMEM((K,), jnp.int32),
        pltpu.VMEM((K,), jnp.float32),
        pltpu.VMEM((2048,), jnp.float32),
        pltpu.VMEM_SHARED((NP,), jnp.float32),
    ],
)
def _deg_kernel(src_hbm, dst_hbm, deg_hbm, idx_v, ones_v, zb_v, cnt_sh):
    c = lax.axis_index("c")
    s = lax.axis_index("s")
    _fill1d(ones_v, K, 1.0)

    @pl.when(s == 0)
    def _():
        _fill1d(zb_v, 2048, 0.0)
        def zc(j, carry):
            pltpu.sync_copy(zb_v, cnt_sh.at[pl.ds(j * 2048, 2048)])
            return carry
        lax.fori_loop(0, NP // 2048, zc, 0)

    plsc.subcore_barrier()

    def count(idx_hbm):
        def body(i, carry):
            off = s * ET + i * K
            pltpu.sync_copy(idx_hbm.at[pl.ds(off, K)], idx_v)
            pltpu.sync_copy(ones_v, cnt_sh.at[idx_v], add=True)
            return carry
        lax.fori_loop(0, NB, body, 0)

    @pl.when(c == 0)
    def _():
        count(src_hbm)

    @pl.when(c == 1)
    def _():
        count(dst_hbm)

    plsc.subcore_barrier()

    @pl.when(s == 0)
    def _():
        def dump(j, carry):
            pltpu.sync_copy(cnt_sh.at[pl.ds(j * 2048, 2048)], zb_v)
            pltpu.sync_copy(zb_v, deg_hbm.at[pl.ds(c * NP + j * 2048, 2048)])
            return carry
        lax.fori_loop(0, NP // 2048, dump, 0)


@functools.partial(
    pl.kernel,
    out_type=(
        jax.ShapeDtypeStruct((NP, HALF), jnp.float32),
        jax.ShapeDtypeStruct((NP, HALF), jnp.float32),
    ),
    mesh=_mesh,
    scratch_types=[
        pltpu.VMEM((K,), jnp.int32),
        pltpu.VMEM((K,), jnp.int32),
        pltpu.VMEM((K, HALF), jnp.float32),
        pltpu.VMEM((16, HALF), jnp.float32),
        pltpu.VMEM_SHARED((NP, HALF), jnp.float32),
        pltpu.SemaphoreType.DMA,
    ],
)
def _scat_kernel(src_hbm, dst_hbm, h0_hbm, h1_hbm, a0_hbm, a1_hbm,
                 sidx, didx, rows, zb, agg_sh, sem):
    c = lax.axis_index("c")
    s = lax.axis_index("s")
    start = s * RPT

    def zrow(j, carry):
        for k2 in range(HALF // LANES):
            zb[j, pl.ds(k2 * LANES, LANES)] = jnp.zeros((LANES,), jnp.float32)
        return carry
    lax.fori_loop(0, 16, zrow, 0)

    def zc(j, carry):
        pltpu.sync_copy(zb, agg_sh.at[pl.ds(start + j * 16, 16)])
        return carry
    lax.fori_loop(0, RPT // 16, zc, 0)

    plsc.subcore_barrier()

    def run(h_hbm):
        def body(i, carry):
            off = s * ET + i * K
            pltpu.sync_copy(src_hbm.at[pl.ds(off, K)], sidx)
            pltpu.sync_copy(dst_hbm.at[pl.ds(off, K)], didx)
            pltpu.async_copy(h_hbm.at[sidx], rows, sem).wait()
            pltpu.sync_copy(rows, agg_sh.at[didx], add=True)
            return carry
        lax.fori_loop(0, NB, body, 0)

    @pl.when(c == 0)
    def _():
        run(h0_hbm)

    @pl.when(c == 1)
    def _():
        run(h1_hbm)

    plsc.subcore_barrier()

    def dump(a_hbm):
        pltpu.sync_copy(agg_sh.at[pl.ds(start, RPT)],
                        a_hbm.at[pl.ds(start, RPT)])

    @pl.when(c == 0)
    def _():
        dump(a0_hbm)

    @pl.when(c == 1)
    def _():
        dump(a1_hbm)


def _scale_body(deg_ref, feat_ref, h0_ref, h1_ref):
    scale = lax.rsqrt(jnp.maximum(deg_ref[...], 1.0))
    hb = feat_ref[...] * scale
    h0_ref[...] = hb[:, :HALF]
    h1_ref[...] = hb[:, HALF:]


_scale_call = pl.pallas_call(
    _scale_body,
    grid=(GRID,),
    in_specs=[
        pl.BlockSpec((BR, 1), lambda i: (i, 0)),
        pl.BlockSpec((BR, D), lambda i: (i, 0)),
    ],
    out_specs=[
        pl.BlockSpec((BR, HALF), lambda i: (i, 0)),
        pl.BlockSpec((BR, HALF), lambda i: (i, 0)),
    ],
    out_shape=[jax.ShapeDtypeStruct((NP, HALF), jnp.float32)] * 2,
)


def _mm_body(deg_ref, w_ref, b_ref, a0_ref, a1_ref, h_ref, stats_ref, acc_ref):
    i = pl.program_id(0)
    sin = lax.rsqrt(jnp.maximum(deg_ref[...], 1.0))
    a = jnp.concatenate([a0_ref[...], a1_ref[...]], axis=1) * sin
    hb = jnp.dot(a, w_ref[...], preferred_element_type=jnp.float32) + b_ref[...]
    h_ref[...] = hb

    rowid = i * BR + lax.broadcasted_iota(jnp.int32, (BR, 1), 0)
    hm = jnp.where(rowid < N, hb, 0.0)

    @pl.when(i == 0)
    def _():
        acc_ref[...] = jnp.zeros_like(acc_ref)

    acc_ref[...] += jnp.stack([jnp.sum(hm, axis=0), jnp.sum(hm * hm, axis=0)])

    @pl.when(i == pl.num_programs(0) - 1)
    def _():
        stats_ref[...] = acc_ref[...]


_mm_call = pl.pallas_call(
    _mm_body,
    grid=(GRID,),
    in_specs=[
        pl.BlockSpec((BR, 1), lambda i: (NP // BR + i, 0)),
        pl.BlockSpec((D, D), lambda i: (0, 0)),
        pl.BlockSpec((1, D), lambda i: (0, 0)),
        pl.BlockSpec((BR, HALF), lambda i: (i, 0)),
        pl.BlockSpec((BR, HALF), lambda i: (i, 0)),
    ],
    out_specs=[
        pl.BlockSpec((BR, D), lambda i: (i, 0)),
        pl.BlockSpec((2, D), lambda i: (0, 0)),
    ],
    out_shape=[
        jax.ShapeDtypeStruct((NP, D), jnp.float32),
        jax.ShapeDtypeStruct((2, D), jnp.float32),
    ],
    scratch_shapes=[pltpu.VMEM((2, D), jnp.float32)],
)


def _bn_body(stats_ref, gamma_ref, beta_ref, h_ref, o_ref):
    mean = stats_ref[0, :] * (1.0 / N)
    ex2 = stats_ref[1, :] * (1.0 / N)
    var = ex2 - mean * mean
    inv = lax.rsqrt(var + BN_EPS)
    o_ref[...] = ((h_ref[...] - mean[None, :]) * (inv * gamma_ref[0, :])[None, :]
                  + beta_ref[0, :][None, :])


_bn_call = pl.pallas_call(
    _bn_body,
    grid=(GRID,),
    in_specs=[
        pl.BlockSpec((2, D), lambda i: (0, 0)),
        pl.BlockSpec((1, D), lambda i: (0, 0)),
        pl.BlockSpec((1, D), lambda i: (0, 0)),
        pl.BlockSpec((BR, D), lambda i: (i, 0)),
    ],
    out_specs=pl.BlockSpec((BR, D), lambda i: (i, 0)),
    out_shape=jax.ShapeDtypeStruct((NP, D), jnp.float32),
)


def kernel(feat, edge_index, W, b, gamma, beta):
    src = edge_index[0].astype(jnp.int32)
    dst = edge_index[1].astype(jnp.int32)
    deg = _deg_kernel(src, dst).reshape(2 * NP, 1)
    h0, h1 = _scale_call(deg, feat)
    a0, a1 = _scat_kernel(src, dst, h0, h1)
    h, stats = _mm_call(deg, W, b.reshape(1, D), a0, a1)
    out = _bn_call(stats, gamma.reshape(1, D), beta.reshape(1, D), h)
    return out[:N]


# speedup vs baseline: 6.4176x; 2.1373x over previous
import functools

import jax
import jax.numpy as jnp
from jax import lax
from jax.experimental import pallas as pl
from jax.experimental.pallas import tpu as pltpu
from jax.experimental.pallas import tpu_sc as plsc

N = 10000
NP = 10240
E = 160000
D = 256
HALF = 128
NC, NS, LANES = 2, 16, 16
ET = E // NS
K = 80
NB = ET // K
RPT = NP // NS
CH = 25
BN_EPS = 1e-5
BR = 2048
GRID = NP // BR

_mesh = plsc.VectorSubcoreMesh(
    core_axis_name="c", subcore_axis_name="s", num_cores=NC, num_subcores=NS
)


def _fill1d(ref, n, val):
    def body(i, carry):
        ref[pl.ds(i * LANES, LANES)] = jnp.full((LANES,), val, jnp.float32)
        return carry
    lax.fori_loop(0, n // LANES, body, 0)


@functools.partial(
    pl.kernel,
    out_type=jax.ShapeDtypeStruct((2 * NP,), jnp.float32),
    mesh=_mesh,
    scratch_types=[
        pltpu.VMEM((NB, K), jnp.int32),
        pltpu.VMEM((K,), jnp.float32),
        pltpu.VMEM((2048,), jnp.float32),
        pltpu.VMEM_SHARED((NP,), jnp.float32),
        pltpu.SemaphoreType.DMA,
    ],
)
def _deg_kernel(src_hbm, dst_hbm, deg_hbm, idx2, ones_v, zb_v, cnt_sh, sem):
    c = lax.axis_index("c")
    s = lax.axis_index("s")
    _fill1d(ones_v, K, 1.0)

    @pl.when(s == 0)
    def _():
        _fill1d(zb_v, 2048, 0.0)
        def zc(j, carry):
            pltpu.sync_copy(zb_v, cnt_sh.at[pl.ds(j * 2048, 2048)])
            return carry
        lax.fori_loop(0, NP // 2048, zc, 0)

    plsc.subcore_barrier()

    @pl.when(c == 0)
    def _():
        pltpu.sync_copy(src_hbm.at[s], idx2)

    @pl.when(c == 1)
    def _():
        pltpu.sync_copy(dst_hbm.at[s], idx2)

    def fire(j, carry):
        pltpu.async_copy(ones_v, cnt_sh.at[idx2.at[j]], sem, add=True)
        return carry
    lax.fori_loop(0, NB, fire, 0)

    def drain(j, carry):
        pltpu.make_async_copy(ones_v, cnt_sh.at[idx2.at[0]], sem).wait()
        return carry
    lax.fori_loop(0, NB, drain, 0)

    plsc.subcore_barrier()

    @pl.when(s == 0)
    def _():
        def dump(j, carry):
            pltpu.sync_copy(cnt_sh.at[pl.ds(j * 2048, 2048)], zb_v)
            pltpu.sync_copy(zb_v, deg_hbm.at[pl.ds(c * NP + j * 2048, 2048)])
            return carry
        lax.fori_loop(0, NP // 2048, dump, 0)


@functools.partial(
    pl.kernel,
    out_type=(
        jax.ShapeDtypeStruct((NP, HALF), jnp.float32),
        jax.ShapeDtypeStruct((NP, HALF), jnp.float32),
    ),
    mesh=_mesh,
    scratch_types=[
        pltpu.VMEM((CH, K), jnp.int32),
        pltpu.VMEM((CH, K), jnp.int32),
        pltpu.VMEM((K, HALF), jnp.float32),
        pltpu.VMEM((K, HALF), jnp.float32),
        pltpu.VMEM((16, HALF), jnp.float32),
        pltpu.VMEM_SHARED((NP, HALF), jnp.float32),
        pltpu.SemaphoreType.DMA,
        pltpu.SemaphoreType.DMA,
    ],
)
def _scat_kernel(src_hbm, dst_hbm, h0_hbm, h1_hbm, a0_hbm, a1_hbm,
                 sidx2, didx2, rows_a, rows_b, zb, agg_sh, sem_a, sem_b):
    c = lax.axis_index("c")
    s = lax.axis_index("s")
    start = s * RPT

    def zrow(j, carry):
        for k2 in range(HALF // LANES):
            zb[j, pl.ds(k2 * LANES, LANES)] = jnp.zeros((LANES,), jnp.float32)
        return carry
    lax.fori_loop(0, 16, zrow, 0)

    def zc(j, carry):
        pltpu.sync_copy(zb, agg_sh.at[pl.ds(start + j * 16, 16)])
        return carry
    lax.fori_loop(0, RPT // 16, zc, 0)

    plsc.subcore_barrier()

    def run(h_hbm):
        def gather(b, rows, sem):
            pltpu.async_copy(h_hbm.at[sidx2.at[b]], rows, sem)

        def wait(rows, sem):
            pltpu.make_async_copy(h_hbm.at[pl.ds(0, K)], rows, sem).wait()

        def scat(b, rows):
            pltpu.sync_copy(rows, agg_sh.at[didx2.at[b]], add=True)

        def chunk(ch, carry):
            pltpu.sync_copy(src_hbm.at[s, ch], sidx2)
            pltpu.sync_copy(dst_hbm.at[s, ch], didx2)
            gather(0, rows_a, sem_a)

            def body(k, c2):
                gather(2 * k + 1, rows_b, sem_b)
                wait(rows_a, sem_a)
                scat(2 * k, rows_a)
                gather(2 * k + 2, rows_a, sem_a)
                wait(rows_b, sem_b)
                scat(2 * k + 1, rows_b)
                return c2
            lax.fori_loop(0, (CH - 1) // 2, body, 0)

            wait(rows_a, sem_a)
            scat(CH - 1, rows_a)
            return carry
        lax.fori_loop(0, NB // CH, chunk, 0)

    @pl.when(c == 0)
    def _():
        run(h0_hbm)

    @pl.when(c == 1)
    def _():
        run(h1_hbm)

    plsc.subcore_barrier()

    def dump(a_hbm):
        pltpu.sync_copy(agg_sh.at[pl.ds(start, RPT)],
                        a_hbm.at[pl.ds(start, RPT)])

    @pl.when(c == 0)
    def _():
        dump(a0_hbm)

    @pl.when(c == 1)
    def _():
        dump(a1_hbm)


def _scale_body(deg_ref, feat_ref, h0_ref, h1_ref):
    scale = lax.rsqrt(jnp.maximum(deg_ref[...], 1.0))
    hb = feat_ref[...] * scale
    h0_ref[...] = hb[:, :HALF]
    h1_ref[...] = hb[:, HALF:]


_scale_call = pl.pallas_call(
    _scale_body,
    grid=(GRID,),
    in_specs=[
        pl.BlockSpec((BR, 1), lambda i: (i, 0)),
        pl.BlockSpec((BR, D), lambda i: (i, 0)),
    ],
    out_specs=[
        pl.BlockSpec((BR, HALF), lambda i: (i, 0)),
        pl.BlockSpec((BR, HALF), lambda i: (i, 0)),
    ],
    out_shape=[jax.ShapeDtypeStruct((NP, HALF), jnp.float32)] * 2,
)


def _mm_body(deg_ref, w_ref, b_ref, a0_ref, a1_ref, h_ref, stats_ref, acc_ref):
    i = pl.program_id(0)
    sin = lax.rsqrt(jnp.maximum(deg_ref[...], 1.0))
    a = jnp.concatenate([a0_ref[...], a1_ref[...]], axis=1) * sin
    hb = jnp.dot(a, w_ref[...], preferred_element_type=jnp.float32) + b_ref[...]
    h_ref[...] = hb

    rowid = i * BR + lax.broadcasted_iota(jnp.int32, (BR, 1), 0)
    hm = jnp.where(rowid < N, hb, 0.0)

    @pl.when(i == 0)
    def _():
        acc_ref[...] = jnp.zeros_like(acc_ref)

    acc_ref[...] += jnp.stack([jnp.sum(hm, axis=0), jnp.sum(hm * hm, axis=0)])

    @pl.when(i == pl.num_programs(0) - 1)
    def _():
        stats_ref[...] = acc_ref[...]


_mm_call = pl.pallas_call(
    _mm_body,
    grid=(GRID,),
    in_specs=[
        pl.BlockSpec((BR, 1), lambda i: (NP // BR + i, 0)),
        pl.BlockSpec((D, D), lambda i: (0, 0)),
        pl.BlockSpec((1, D), lambda i: (0, 0)),
        pl.BlockSpec((BR, HALF), lambda i: (i, 0)),
        pl.BlockSpec((BR, HALF), lambda i: (i, 0)),
    ],
    out_specs=[
        pl.BlockSpec((BR, D), lambda i: (i, 0)),
        pl.BlockSpec((2, D), lambda i: (0, 0)),
    ],
    out_shape=[
        jax.ShapeDtypeStruct((NP, D), jnp.float32),
        jax.ShapeDtypeStruct((2, D), jnp.float32),
    ],
    scratch_shapes=[pltpu.VMEM((2, D), jnp.float32)],
)


def _bn_body(stats_ref, gamma_ref, beta_ref, h_ref, o_ref):
    mean = stats_ref[0, :] * (1.0 / N)
    ex2 = stats_ref[1, :] * (1.0 / N)
    var = ex2 - mean * mean
    inv = lax.rsqrt(var + BN_EPS)
    o_ref[...] = ((h_ref[...] - mean[None, :]) * (inv * gamma_ref[0, :])[None, :]
                  + beta_ref[0, :][None, :])


_bn_call = pl.pallas_call(
    _bn_body,
    grid=(GRID,),
    in_specs=[
        pl.BlockSpec((2, D), lambda i: (0, 0)),
        pl.BlockSpec((1, D), lambda i: (0, 0)),
        pl.BlockSpec((1, D), lambda i: (0, 0)),
        pl.BlockSpec((BR, D), lambda i: (i, 0)),
    ],
    out_specs=pl.BlockSpec((BR, D), lambda i: (i, 0)),
    out_shape=jax.ShapeDtypeStruct((NP, D), jnp.float32),
)


def kernel(feat, edge_index, W, b, gamma, beta):
    src = edge_index[0].astype(jnp.int32).reshape(NS, NB, K)
    dst = edge_index[1].astype(jnp.int32).reshape(NS, NB, K)
    deg = _deg_kernel(src, dst).reshape(2 * NP, 1)
    h0, h1 = _scale_call(deg, feat)
    a0, a1 = _scat_kernel(src.reshape(NS, NB // CH, CH, K),
                          dst.reshape(NS, NB // CH, CH, K), h0, h1)
    h, stats = _mm_call(deg, W, b.reshape(1, D), a0, a1)
    out = _bn_call(stats, gamma.reshape(1, D), beta.reshape(1, D), h)
    return out[:N]


# speedup vs baseline: 6.6545x; 1.0369x over previous
import functools

import jax
import jax.numpy as jnp
from jax import lax
from jax.experimental import pallas as pl
from jax.experimental.pallas import tpu as pltpu
from jax.experimental.pallas import tpu_sc as plsc

N = 10000
NP = 10240
E = 160000
D = 256
HALF = 128
NC, NS, LANES = 2, 16, 16
ET = E // NS
K = 80
NB = ET // K
KS = 80
CHS = 25
NCH = ET // (KS * CHS)
RPT = NP // NS
BN_EPS = 1e-5
BR = 2048
GRID = NP // BR

_mesh = plsc.VectorSubcoreMesh(
    core_axis_name="c", subcore_axis_name="s", num_cores=NC, num_subcores=NS
)


def _fill1d(ref, n, val):
    def body(i, carry):
        ref[pl.ds(i * LANES, LANES)] = jnp.full((LANES,), val, jnp.float32)
        return carry
    lax.fori_loop(0, n // LANES, body, 0)


@functools.partial(
    pl.kernel,
    out_type=jax.ShapeDtypeStruct((2 * NP,), jnp.float32),
    mesh=_mesh,
    scratch_types=[
        pltpu.VMEM((NB, K), jnp.int32),
        pltpu.VMEM((K,), jnp.float32),
        pltpu.VMEM((2048,), jnp.float32),
        pltpu.VMEM_SHARED((NP,), jnp.float32),
        pltpu.SemaphoreType.DMA,
    ],
)
def _deg_kernel(src_hbm, dst_hbm, deg_hbm, idx2, ones_v, zb_v, cnt_sh, sem):
    c = lax.axis_index("c")
    s = lax.axis_index("s")
    _fill1d(ones_v, K, 1.0)

    @pl.when(s == 0)
    def _():
        _fill1d(zb_v, 2048, 0.0)
        def zc(j, carry):
            pltpu.sync_copy(zb_v, cnt_sh.at[pl.ds(j * 2048, 2048)])
            return carry
        lax.fori_loop(0, NP // 2048, zc, 0)

    plsc.subcore_barrier()

    @pl.when(c == 0)
    def _():
        pltpu.sync_copy(src_hbm.at[s], idx2)

    @pl.when(c == 1)
    def _():
        pltpu.sync_copy(dst_hbm.at[s], idx2)

    def fire(j, carry):
        pltpu.async_copy(ones_v, cnt_sh.at[idx2.at[j]], sem, add=True)
        return carry
    lax.fori_loop(0, NB, fire, 0)

    def drain(j, carry):
        pltpu.make_async_copy(ones_v, cnt_sh.at[idx2.at[0]], sem).wait()
        return carry
    lax.fori_loop(0, NB, drain, 0)

    plsc.subcore_barrier()

    @pl.when(s == 0)
    def _():
        def dump(j, carry):
            pltpu.sync_copy(cnt_sh.at[pl.ds(j * 2048, 2048)], zb_v)
            pltpu.sync_copy(zb_v, deg_hbm.at[pl.ds(c * NP + j * 2048, 2048)])
            return carry
        lax.fori_loop(0, NP // 2048, dump, 0)


NSLOT = 4


@functools.partial(
    pl.kernel,
    out_type=(
        jax.ShapeDtypeStruct((NP, HALF), jnp.float32),
        jax.ShapeDtypeStruct((NP, HALF), jnp.float32),
    ),
    mesh=_mesh,
    scratch_types=(
        [pltpu.VMEM((CHS, KS), jnp.int32)] * 2 +
        [pltpu.VMEM((KS, HALF), jnp.float32)] * NSLOT +
        [pltpu.VMEM_SHARED((NP, HALF), jnp.float32)] +
        [pltpu.SemaphoreType.DMA] * (2 * NSLOT)
    ),
)
def _scat_kernel(src_hbm, dst_hbm, h0_hbm, h1_hbm, a0_hbm, a1_hbm,
                 sidx2, didx2, r0, r1, r2, r3, agg_sh,
                 g0, g1, g2, g3, s0, s1, s2, s3):
    c = lax.axis_index("c")
    s = lax.axis_index("s")
    start = s * RPT
    rows = (r0, r1, r2, r3)
    gsem = (g0, g1, g2, g3)
    ssem = (s0, s1, s2, s3)

    def zrow(j, carry):
        for k2 in range(HALF // LANES):
            r0[j, pl.ds(k2 * LANES, LANES)] = jnp.zeros((LANES,), jnp.float32)
        return carry
    lax.fori_loop(0, KS, zrow, 0)

    def zc(j, carry):
        pltpu.sync_copy(r0, agg_sh.at[pl.ds(start + j * KS, KS)])
        return carry
    lax.fori_loop(0, RPT // KS, zc, 0)

    plsc.subcore_barrier()

    def run(h_hbm):
        def gather(b, u):
            pltpu.async_copy(h_hbm.at[sidx2.at[b]], rows[u], gsem[u])

        def wait_g(u):
            pltpu.make_async_copy(h_hbm.at[pl.ds(0, KS)], rows[u],
                                  gsem[u]).wait()

        def scat(b, u):
            pltpu.async_copy(rows[u], agg_sh.at[didx2.at[b]], ssem[u],
                             add=True)

        def wait_s(u):
            pltpu.make_async_copy(rows[u], agg_sh.at[didx2.at[0]],
                                  ssem[u]).wait()

        def chunk(ch, carry):
            pltpu.sync_copy(src_hbm.at[s, ch], sidx2)
            pltpu.sync_copy(dst_hbm.at[s, ch], didx2)
            for u in range(NSLOT):
                gather(u, u)

            def body(k, c2):
                for u in range(NSLOT):
                    wait_g(u)
                    scat(NSLOT * k + u, u)
                for u in range(NSLOT):
                    wait_s(u)
                    gather(NSLOT * k + u + NSLOT, u)
                return c2
            nf = (CHS - NSLOT) // NSLOT
            lax.fori_loop(0, nf, body, 0)

            wait_g(0)
            scat(NSLOT * nf, 0)
            wait_s(0)
            gather(CHS - 1, 0)
            for u in range(1, NSLOT):
                wait_g(u)
                scat(NSLOT * nf + u, u)
            wait_g(0)
            scat(CHS - 1, 0)
            for u in range(NSLOT):
                wait_s(u)
            return carry
        lax.fori_loop(0, NCH, chunk, 0)

    @pl.when(c == 0)
    def _():
        run(h0_hbm)

    @pl.when(c == 1)
    def _():
        run(h1_hbm)

    plsc.subcore_barrier()

    def dump(a_hbm):
        pltpu.sync_copy(agg_sh.at[pl.ds(start, RPT)],
                        a_hbm.at[pl.ds(start, RPT)])

    @pl.when(c == 0)
    def _():
        dump(a0_hbm)

    @pl.when(c == 1)
    def _():
        dump(a1_hbm)


def _scale_body(deg_ref, feat_ref, h0_ref, h1_ref):
    scale = lax.rsqrt(jnp.maximum(deg_ref[...], 1.0))
    hb = feat_ref[...] * scale
    h0_ref[...] = hb[:, :HALF]
    h1_ref[...] = hb[:, HALF:]


_scale_call = pl.pallas_call(
    _scale_body,
    grid=(GRID,),
    in_specs=[
        pl.BlockSpec((BR, 1), lambda i: (i, 0)),
        pl.BlockSpec((BR, D), lambda i: (i, 0)),
    ],
    out_specs=[
        pl.BlockSpec((BR, HALF), lambda i: (i, 0)),
        pl.BlockSpec((BR, HALF), lambda i: (i, 0)),
    ],
    out_shape=[jax.ShapeDtypeStruct((NP, HALF), jnp.float32)] * 2,
)


def _mm_body(deg_ref, w_ref, b_ref, a0_ref, a1_ref, h_ref, stats_ref, acc_ref):
    i = pl.program_id(0)
    sin = lax.rsqrt(jnp.maximum(deg_ref[...], 1.0))
    a = jnp.concatenate([a0_ref[...], a1_ref[...]], axis=1) * sin
    hb = jnp.dot(a, w_ref[...], preferred_element_type=jnp.float32) + b_ref[...]
    h_ref[...] = hb

    rowid = i * BR + lax.broadcasted_iota(jnp.int32, (BR, 1), 0)
    hm = jnp.where(rowid < N, hb, 0.0)

    @pl.when(i == 0)
    def _():
        acc_ref[...] = jnp.zeros_like(acc_ref)

    acc_ref[...] += jnp.stack([jnp.sum(hm, axis=0), jnp.sum(hm * hm, axis=0)])

    @pl.when(i == pl.num_programs(0) - 1)
    def _():
        stats_ref[...] = acc_ref[...]


_mm_call = pl.pallas_call(
    _mm_body,
    grid=(GRID,),
    in_specs=[
        pl.BlockSpec((BR, 1), lambda i: (NP // BR + i, 0)),
        pl.BlockSpec((D, D), lambda i: (0, 0)),
        pl.BlockSpec((1, D), lambda i: (0, 0)),
        pl.BlockSpec((BR, HALF), lambda i: (i, 0)),
        pl.BlockSpec((BR, HALF), lambda i: (i, 0)),
    ],
    out_specs=[
        pl.BlockSpec((BR, D), lambda i: (i, 0)),
        pl.BlockSpec((2, D), lambda i: (0, 0)),
    ],
    out_shape=[
        jax.ShapeDtypeStruct((NP, D), jnp.float32),
        jax.ShapeDtypeStruct((2, D), jnp.float32),
    ],
    scratch_shapes=[pltpu.VMEM((2, D), jnp.float32)],
)


def _bn_body(stats_ref, gamma_ref, beta_ref, h_ref, o_ref):
    mean = stats_ref[0, :] * (1.0 / N)
    ex2 = stats_ref[1, :] * (1.0 / N)
    var = ex2 - mean * mean
    inv = lax.rsqrt(var + BN_EPS)
    o_ref[...] = ((h_ref[...] - mean[None, :]) * (inv * gamma_ref[0, :])[None, :]
                  + beta_ref[0, :][None, :])


_bn_call = pl.pallas_call(
    _bn_body,
    grid=(GRID,),
    in_specs=[
        pl.BlockSpec((2, D), lambda i: (0, 0)),
        pl.BlockSpec((1, D), lambda i: (0, 0)),
        pl.BlockSpec((1, D), lambda i: (0, 0)),
        pl.BlockSpec((BR, D), lambda i: (i, 0)),
    ],
    out_specs=pl.BlockSpec((BR, D), lambda i: (i, 0)),
    out_shape=jax.ShapeDtypeStruct((NP, D), jnp.float32),
)


def kernel(feat, edge_index, W, b, gamma, beta):
    src = edge_index[0].astype(jnp.int32).reshape(NS, NB, K)
    dst = edge_index[1].astype(jnp.int32).reshape(NS, NB, K)
    deg = _deg_kernel(src, dst).reshape(2 * NP, 1)
    h0, h1 = _scale_call(deg, feat)
    a0, a1 = _scat_kernel(src.reshape(NS, NCH, CHS, KS),
                          dst.reshape(NS, NCH, CHS, KS), h0, h1)
    h, stats = _mm_call(deg, W, b.reshape(1, D), a0, a1)
    out = _bn_call(stats, gamma.reshape(1, D), beta.reshape(1, D), h)
    return out[:N]


# speedup vs baseline: 6.7725x; 1.0177x over previous
import functools

import jax
import jax.numpy as jnp
from jax import lax
from jax.experimental import pallas as pl
from jax.experimental.pallas import tpu as pltpu
from jax.experimental.pallas import tpu_sc as plsc

N = 10000
NP = 10240
E = 160000
D = 256
HALF = 128
NC, NS, LANES = 2, 16, 16
ET = E // NS
K = 80
NB = ET // K
KS = 80
CHS = 25
NCH = ET // (KS * CHS)
RPT = NP // NS
BN_EPS = 1e-5
BR = 2048
GRID = NP // BR

_mesh = plsc.VectorSubcoreMesh(
    core_axis_name="c", subcore_axis_name="s", num_cores=NC, num_subcores=NS
)


def _fill1d(ref, n, val):
    def body(i, carry):
        ref[pl.ds(i * LANES, LANES)] = jnp.full((LANES,), val, jnp.float32)
        return carry
    lax.fori_loop(0, n // LANES, body, 0)


@functools.partial(
    pl.kernel,
    out_type=jax.ShapeDtypeStruct((2 * NP,), jnp.float32),
    mesh=_mesh,
    scratch_types=[
        pltpu.VMEM((NB, K), jnp.int32),
        pltpu.VMEM((K,), jnp.float32),
        pltpu.VMEM((2048,), jnp.float32),
        pltpu.VMEM_SHARED((NP,), jnp.float32),
        pltpu.SemaphoreType.DMA,
    ],
)
def _deg_kernel(src_hbm, dst_hbm, deg_hbm, idx2, ones_v, zb_v, cnt_sh, sem):
    c = lax.axis_index("c")
    s = lax.axis_index("s")
    _fill1d(ones_v, K, 1.0)

    @pl.when(s == 0)
    def _():
        _fill1d(zb_v, 2048, 0.0)
        def zc(j, carry):
            pltpu.sync_copy(zb_v, cnt_sh.at[pl.ds(j * 2048, 2048)])
            return carry
        lax.fori_loop(0, NP // 2048, zc, 0)

    plsc.subcore_barrier()

    @pl.when(c == 0)
    def _():
        pltpu.sync_copy(src_hbm.at[s], idx2)

    @pl.when(c == 1)
    def _():
        pltpu.sync_copy(dst_hbm.at[s], idx2)

    def fire(j, carry):
        pltpu.async_copy(ones_v, cnt_sh.at[idx2.at[j]], sem, add=True)
        return carry
    lax.fori_loop(0, NB, fire, 0)

    def drain(j, carry):
        pltpu.make_async_copy(ones_v, cnt_sh.at[idx2.at[0]], sem).wait()
        return carry
    lax.fori_loop(0, NB, drain, 0)

    plsc.subcore_barrier()

    @pl.when(s == 0)
    def _():
        def dump(j, carry):
            pltpu.sync_copy(cnt_sh.at[pl.ds(j * 2048, 2048)], zb_v)
            pltpu.sync_copy(zb_v, deg_hbm.at[pl.ds(c * NP + j * 2048, 2048)])
            return carry
        lax.fori_loop(0, NP // 2048, dump, 0)


NSLOT = 4


@functools.partial(
    pl.kernel,
    out_type=(
        jax.ShapeDtypeStruct((NP, HALF), jnp.float32),
        jax.ShapeDtypeStruct((NP, HALF), jnp.float32),
    ),
    mesh=_mesh,
    scratch_types=(
        [pltpu.VMEM((CHS, KS), jnp.int32)] * 2 +
        [pltpu.VMEM((KS, HALF), jnp.float32)] * NSLOT +
        [pltpu.VMEM_SHARED((NP, HALF), jnp.float32)] +
        [pltpu.SemaphoreType.DMA] * (2 * NSLOT)
    ),
)
def _scat_kernel(src_hbm, dst_hbm, h0_hbm, h1_hbm, a0_hbm, a1_hbm,
                 sidx2, didx2, r0, r1, r2, r3, agg_sh,
                 g0, g1, g2, g3, s0, s1, s2, s3):
    c = lax.axis_index("c")
    s = lax.axis_index("s")
    start = s * RPT
    rows = (r0, r1, r2, r3)
    gsem = (g0, g1, g2, g3)
    ssem = (s0, s1, s2, s3)

    def zrow(j, carry):
        for k2 in range(HALF // LANES):
            r0[j, pl.ds(k2 * LANES, LANES)] = jnp.zeros((LANES,), jnp.float32)
        return carry
    lax.fori_loop(0, KS, zrow, 0)

    def zc(j, carry):
        pltpu.sync_copy(r0, agg_sh.at[pl.ds(start + j * KS, KS)])
        return carry
    lax.fori_loop(0, RPT // KS, zc, 0)

    plsc.subcore_barrier()

    def run(h_hbm):
        def gather(b, u):
            pltpu.async_copy(h_hbm.at[sidx2.at[b]], rows[u], gsem[u])

        def wait_g(u):
            pltpu.make_async_copy(h_hbm.at[pl.ds(0, KS)], rows[u],
                                  gsem[u]).wait()

        def scat(b, u):
            pltpu.async_copy(rows[u], agg_sh.at[didx2.at[b]], ssem[u],
                             add=True)

        def wait_s(u):
            pltpu.make_async_copy(rows[u], agg_sh.at[didx2.at[0]],
                                  ssem[u]).wait()

        def chunk(ch, carry):
            pltpu.sync_copy(src_hbm.at[s, ch], sidx2)
            pltpu.sync_copy(dst_hbm.at[s, ch], didx2)
            for u in range(NSLOT):
                gather(u, u)

            def body(k, c2):
                for u in range(NSLOT):
                    wait_g(u)
                    scat(NSLOT * k + u, u)
                for u in range(NSLOT):
                    wait_s(u)
                    gather(NSLOT * k + u + NSLOT, u)
                return c2
            nf = (CHS - NSLOT) // NSLOT
            lax.fori_loop(0, nf, body, 0)

            wait_g(0)
            scat(NSLOT * nf, 0)
            wait_s(0)
            gather(CHS - 1, 0)
            for u in range(1, NSLOT):
                wait_g(u)
                scat(NSLOT * nf + u, u)
            wait_g(0)
            scat(CHS - 1, 0)
            for u in range(NSLOT):
                wait_s(u)
            return carry
        lax.fori_loop(0, NCH, chunk, 0)

    @pl.when(c == 0)
    def _():
        run(h0_hbm)

    @pl.when(c == 1)
    def _():
        run(h1_hbm)

    plsc.subcore_barrier()

    def dump(a_hbm):
        pltpu.sync_copy(agg_sh.at[pl.ds(start, RPT)],
                        a_hbm.at[pl.ds(start, RPT)])

    @pl.when(c == 0)
    def _():
        dump(a0_hbm)

    @pl.when(c == 1)
    def _():
        dump(a1_hbm)


def _scale_body(deg_ref, feat_ref, h0_ref, h1_ref):
    scale = lax.rsqrt(jnp.maximum(deg_ref[...], 1.0))
    hb = feat_ref[...] * scale
    h0_ref[...] = hb[:, :HALF]
    h1_ref[...] = hb[:, HALF:]


_scale_call = pl.pallas_call(
    _scale_body,
    grid=(GRID,),
    in_specs=[
        pl.BlockSpec((BR, 1), lambda i: (i, 0)),
        pl.BlockSpec((BR, D), lambda i: (i, 0)),
    ],
    out_specs=[
        pl.BlockSpec((BR, HALF), lambda i: (i, 0)),
        pl.BlockSpec((BR, HALF), lambda i: (i, 0)),
    ],
    out_shape=[jax.ShapeDtypeStruct((NP, HALF), jnp.float32)] * 2,
)


def _mm_bn_body(deg_ref, w_ref, b_ref, gamma_ref, beta_ref, a0_ref, a1_ref,
                o_ref, hbuf_ref, acc_ref):
    p = pl.program_id(0)
    i = pl.program_id(1)

    @pl.when(p == 0)
    def _():
        sin = lax.rsqrt(jnp.maximum(deg_ref[...], 1.0))
        a = jnp.concatenate([a0_ref[...], a1_ref[...]], axis=1) * sin
        hb = (jnp.dot(a, w_ref[...], preferred_element_type=jnp.float32)
              + b_ref[...])
        hbuf_ref[pl.ds(i * BR, BR), :] = hb
        o_ref[...] = hb

        rowid = i * BR + lax.broadcasted_iota(jnp.int32, (BR, 1), 0)
        hm = jnp.where(rowid < N, hb, 0.0)

        @pl.when(i == 0)
        def _():
            acc_ref[...] = jnp.zeros_like(acc_ref)

        acc_ref[...] += jnp.stack([jnp.sum(hm, axis=0),
                                   jnp.sum(hm * hm, axis=0)])

    @pl.when(p == 1)
    def _():
        mean = acc_ref[0, :] * (1.0 / N)
        ex2 = acc_ref[1, :] * (1.0 / N)
        var = ex2 - mean * mean
        inv = lax.rsqrt(var + BN_EPS)
        hb = hbuf_ref[pl.ds(i * BR, BR), :]
        o_ref[...] = ((hb - mean[None, :]) * (inv * gamma_ref[0, :])[None, :]
                      + beta_ref[0, :][None, :])


_mm_bn_call = pl.pallas_call(
    _mm_bn_body,
    grid=(2, GRID),
    in_specs=[
        pl.BlockSpec((BR, 1), lambda p, i: (NP // BR + i, 0)),
        pl.BlockSpec((D, D), lambda p, i: (0, 0)),
        pl.BlockSpec((1, D), lambda p, i: (0, 0)),
        pl.BlockSpec((1, D), lambda p, i: (0, 0)),
        pl.BlockSpec((1, D), lambda p, i: (0, 0)),
        pl.BlockSpec((BR, HALF), lambda p, i: (i * (1 - p), 0)),
        pl.BlockSpec((BR, HALF), lambda p, i: (i * (1 - p), 0)),
    ],
    out_specs=pl.BlockSpec((BR, D), lambda p, i: (i, 0)),
    out_shape=jax.ShapeDtypeStruct((NP, D), jnp.float32),
    scratch_shapes=[
        pltpu.VMEM((NP, D), jnp.float32),
        pltpu.VMEM((2, D), jnp.float32),
    ],
)


def kernel(feat, edge_index, W, b, gamma, beta):
    src = edge_index[0].astype(jnp.int32).reshape(NS, NB, K)
    dst = edge_index[1].astype(jnp.int32).reshape(NS, NB, K)
    deg = _deg_kernel(src, dst).reshape(2 * NP, 1)
    h0, h1 = _scale_call(deg, feat)
    a0, a1 = _scat_kernel(src.reshape(NS, NCH, CHS, KS),
                          dst.reshape(NS, NCH, CHS, KS), h0, h1)
    out = _mm_bn_call(deg, W, b.reshape(1, D), gamma.reshape(1, D),
                      beta.reshape(1, D), a0, a1)
    return out[:N]


# speedup vs baseline: 6.8410x; 1.0101x over previous
import functools

import jax
import jax.numpy as jnp
from jax import lax
from jax.experimental import pallas as pl
from jax.experimental.pallas import tpu as pltpu
from jax.experimental.pallas import tpu_sc as plsc

N = 10000
NP = 10240
E = 160000
D = 256
HALF = 128
NC, NS, LANES = 2, 16, 16
ET = E // NS
K = 80
NB = ET // K
KS = 80
CHS = 25
NCH = ET // (KS * CHS)
RPT = NP // NS
BN_EPS = 1e-5
BR = 2048
GRID = NP // BR

_mesh = plsc.VectorSubcoreMesh(
    core_axis_name="c", subcore_axis_name="s", num_cores=NC, num_subcores=NS
)


def _fill1d(ref, n, val):
    def body(i, carry):
        ref[pl.ds(i * LANES, LANES)] = jnp.full((LANES,), val, jnp.float32)
        return carry
    lax.fori_loop(0, n // LANES, body, 0)


@functools.partial(
    pl.kernel,
    out_type=jax.ShapeDtypeStruct((2 * NP,), jnp.float32),
    mesh=_mesh,
    scratch_types=[
        pltpu.VMEM((NB, K), jnp.int32),
        pltpu.VMEM((K,), jnp.float32),
        pltpu.VMEM((2048,), jnp.float32),
        pltpu.VMEM_SHARED((NP,), jnp.float32),
        pltpu.SemaphoreType.DMA,
    ],
)
def _deg_kernel(src_hbm, dst_hbm, deg_hbm, idx2, ones_v, zb_v, cnt_sh, sem):
    c = lax.axis_index("c")
    s = lax.axis_index("s")
    _fill1d(ones_v, K, 1.0)

    @pl.when(s == 0)
    def _():
        _fill1d(zb_v, 2048, 0.0)
        def zc(j, carry):
            pltpu.sync_copy(zb_v, cnt_sh.at[pl.ds(j * 2048, 2048)])
            return carry
        lax.fori_loop(0, NP // 2048, zc, 0)

    plsc.subcore_barrier()

    @pl.when(c == 0)
    def _():
        pltpu.sync_copy(src_hbm.at[s], idx2)

    @pl.when(c == 1)
    def _():
        pltpu.sync_copy(dst_hbm.at[s], idx2)

    def fire(j, carry):
        pltpu.async_copy(ones_v, cnt_sh.at[idx2.at[j]], sem, add=True)
        return carry
    lax.fori_loop(0, NB, fire, 0)

    def drain(j, carry):
        pltpu.make_async_copy(ones_v, cnt_sh.at[idx2.at[0]], sem).wait()
        return carry
    lax.fori_loop(0, NB, drain, 0)

    plsc.subcore_barrier()

    @pl.when(s == 0)
    def _():
        def dump(j, carry):
            pltpu.sync_copy(cnt_sh.at[pl.ds(j * 2048, 2048)], zb_v)
            pltpu.sync_copy(zb_v, deg_hbm.at[pl.ds(c * NP + j * 2048, 2048)])
            return carry
        lax.fori_loop(0, NP // 2048, dump, 0)


NSLOT = 4


@functools.partial(
    pl.kernel,
    out_type=(
        jax.ShapeDtypeStruct((NP, HALF), jnp.float32),
        jax.ShapeDtypeStruct((NP, HALF), jnp.float32),
    ),
    mesh=_mesh,
    scratch_types=(
        [pltpu.VMEM((CHS, KS), jnp.int32)] * 2 +
        [pltpu.VMEM((KS, HALF), jnp.float32)] * NSLOT +
        [pltpu.VMEM_SHARED((NP, HALF), jnp.float32)] +
        [pltpu.SemaphoreType.DMA] * (2 * NSLOT)
    ),
)
def _scat_kernel(src_hbm, dst_hbm, h0_hbm, h1_hbm, a0_hbm, a1_hbm,
                 sidx2, didx2, r0, r1, r2, r3, agg_sh,
                 g0, g1, g2, g3, s0, s1, s2, s3):
    c = lax.axis_index("c")
    s = lax.axis_index("s")
    start = s * RPT
    rows = (r0, r1, r2, r3)
    gsem = (g0, g1, g2, g3)
    ssem = (s0, s1, s2, s3)

    def zrow(j, carry):
        for k2 in range(HALF // LANES):
            r0[j, pl.ds(k2 * LANES, LANES)] = jnp.zeros((LANES,), jnp.float32)
        return carry
    lax.fori_loop(0, KS, zrow, 0)

    def zc(j, carry):
        pltpu.sync_copy(r0, agg_sh.at[pl.ds(start + j * KS, KS)])
        return carry
    lax.fori_loop(0, RPT // KS, zc, 0)

    plsc.subcore_barrier()

    def run(h_hbm):
        def gather(b, u):
            pltpu.async_copy(h_hbm.at[sidx2.at[b]], rows[u], gsem[u])

        def wait_g(u):
            pltpu.make_async_copy(h_hbm.at[pl.ds(0, KS)], rows[u],
                                  gsem[u]).wait()

        def scat(b, u):
            pltpu.async_copy(rows[u], agg_sh.at[didx2.at[b]], ssem[u],
                             add=True)

        def wait_s(u):
            pltpu.make_async_copy(rows[u], agg_sh.at[didx2.at[0]],
                                  ssem[u]).wait()

        def chunk(ch, carry):
            pltpu.sync_copy(src_hbm.at[s, ch], sidx2)
            pltpu.sync_copy(dst_hbm.at[s, ch], didx2)
            for u in range(NSLOT):
                gather(u, u)

            def body(k, c2):
                for u in range(NSLOT):
                    wait_g(u)
                    scat(NSLOT * k + u, u)
                for u in range(NSLOT):
                    wait_s(u)
                    gather(NSLOT * k + u + NSLOT, u)
                return c2
            nf = (CHS - NSLOT) // NSLOT
            lax.fori_loop(0, nf, body, 0)

            wait_g(0)
            scat(NSLOT * nf, 0)
            wait_s(0)
            gather(CHS - 1, 0)
            for u in range(1, NSLOT):
                wait_g(u)
                scat(NSLOT * nf + u, u)
            wait_g(0)
            scat(CHS - 1, 0)
            for u in range(NSLOT):
                wait_s(u)
            return carry
        lax.fori_loop(0, NCH, chunk, 0)

    @pl.when(c == 0)
    def _():
        run(h0_hbm)

    @pl.when(c == 1)
    def _():
        run(h1_hbm)

    plsc.subcore_barrier()

    def dump(a_hbm):
        pltpu.sync_copy(agg_sh.at[pl.ds(start, RPT)],
                        a_hbm.at[pl.ds(start, RPT)])

    @pl.when(c == 0)
    def _():
        dump(a0_hbm)

    @pl.when(c == 1)
    def _():
        dump(a1_hbm)


def _scale_body(deg_ref, feat_ref, h0_ref, h1_ref):
    scale = lax.rsqrt(jnp.maximum(deg_ref[...], 1.0))
    hb = feat_ref[...] * scale
    h0_ref[...] = hb[:, :HALF]
    h1_ref[...] = hb[:, HALF:]


_scale_call = pl.pallas_call(
    _scale_body,
    grid=(GRID,),
    in_specs=[
        pl.BlockSpec((BR, 1), lambda i: (i, 0)),
        pl.BlockSpec((BR, D), lambda i: (i, 0)),
    ],
    out_specs=[
        pl.BlockSpec((BR, HALF), lambda i: (i, 0)),
        pl.BlockSpec((BR, HALF), lambda i: (i, 0)),
    ],
    out_shape=[jax.ShapeDtypeStruct((NP, HALF), jnp.float32)] * 2,
)


def _mm_bn_body(deg_ref, w_ref, b_ref, gamma_ref, beta_ref, a0_ref, a1_ref,
                o_ref, hbuf_ref, acc_ref):
    p = pl.program_id(0)
    i = pl.program_id(1)

    @pl.when(p == 0)
    def _():
        sin = lax.rsqrt(jnp.maximum(deg_ref[...], 1.0))
        a = jnp.concatenate([a0_ref[...], a1_ref[...]], axis=1) * sin
        hb = (jnp.dot(a, w_ref[...], preferred_element_type=jnp.float32)
              + b_ref[...])
        hbuf_ref[pl.ds(i * BR, BR), :] = hb
        o_ref[...] = hb

        rowid = i * BR + lax.broadcasted_iota(jnp.int32, (BR, 1), 0)
        hm = jnp.where(rowid < N, hb, 0.0)

        @pl.when(i == 0)
        def _():
            acc_ref[...] = jnp.zeros_like(acc_ref)

        acc_ref[...] += jnp.stack([jnp.sum(hm, axis=0),
                                   jnp.sum(hm * hm, axis=0)])

    @pl.when(p == 1)
    def _():
        mean = acc_ref[0, :] * (1.0 / N)
        ex2 = acc_ref[1, :] * (1.0 / N)
        var = ex2 - mean * mean
        inv = lax.rsqrt(var + BN_EPS)
        hb = hbuf_ref[pl.ds(i * BR, BR), :]
        o_ref[...] = ((hb - mean[None, :]) * (inv * gamma_ref[0, :])[None, :]
                      + beta_ref[0, :][None, :])


_mm_bn_call = pl.pallas_call(
    _mm_bn_body,
    grid=(2, GRID),
    in_specs=[
        pl.BlockSpec((BR, 1), lambda p, i: (NP // BR + i, 0)),
        pl.BlockSpec((D, D), lambda p, i: (0, 0)),
        pl.BlockSpec((1, D), lambda p, i: (0, 0)),
        pl.BlockSpec((1, D), lambda p, i: (0, 0)),
        pl.BlockSpec((1, D), lambda p, i: (0, 0)),
        pl.BlockSpec((BR, HALF), lambda p, i: (i * (1 - p), 0)),
        pl.BlockSpec((BR, HALF), lambda p, i: (i * (1 - p), 0)),
    ],
    out_specs=pl.BlockSpec((BR, D), lambda p, i: (i * p, 0)),
    out_shape=jax.ShapeDtypeStruct((NP, D), jnp.float32),
    scratch_shapes=[
        pltpu.VMEM((NP, D), jnp.float32),
        pltpu.VMEM((2, D), jnp.float32),
    ],
)


def kernel(feat, edge_index, W, b, gamma, beta):
    src = edge_index[0].astype(jnp.int32).reshape(NS, NB, K)
    dst = edge_index[1].astype(jnp.int32).reshape(NS, NB, K)
    deg = _deg_kernel(src, dst).reshape(2 * NP, 1)
    h0, h1 = _scale_call(deg, feat)
    a0, a1 = _scat_kernel(src.reshape(NS, NCH, CHS, KS),
                          dst.reshape(NS, NCH, CHS, KS), h0, h1)
    out = _mm_bn_call(deg, W, b.reshape(1, D), gamma.reshape(1, D),
                      beta.reshape(1, D), a0, a1)
    return out[:N]


# speedup vs baseline: 7.3968x; 1.0812x over previous
import functools

import jax
import jax.numpy as jnp
from jax import lax
from jax.experimental import pallas as pl
from jax.experimental.pallas import tpu as pltpu
from jax.experimental.pallas import tpu_sc as plsc

N = 10000
NP = 10240
E = 160000
D = 256
HALF = 128
NC, NS, LANES = 2, 16, 16
ET = E // NS
K = 80
NB = ET // K
KS = 80
CHS = 25
NCH = ET // (KS * CHS)
RPT = NP // NS
BN_EPS = 1e-5
BR = 2048
GRID = NP // BR

_mesh = plsc.VectorSubcoreMesh(
    core_axis_name="c", subcore_axis_name="s", num_cores=NC, num_subcores=NS
)


def _fill1d(ref, n, val):
    def body(i, carry):
        ref[pl.ds(i * LANES, LANES)] = jnp.full((LANES,), val, jnp.float32)
        return carry
    lax.fori_loop(0, n // LANES, body, 0)


@functools.partial(
    pl.kernel,
    out_type=jax.ShapeDtypeStruct((2 * NP,), jnp.float32),
    mesh=_mesh,
    scratch_types=[
        pltpu.VMEM((NB, K), jnp.int32),
        pltpu.VMEM((K,), jnp.float32),
        pltpu.VMEM((2048,), jnp.float32),
        pltpu.VMEM_SHARED((NP,), jnp.float32),
        pltpu.SemaphoreType.DMA,
    ],
)
def _deg_kernel(ei_hbm, deg_hbm, idx2, ones_v, zb_v, cnt_sh, sem):
    c = lax.axis_index("c")
    s = lax.axis_index("s")
    _fill1d(ones_v, K, 1.0)

    @pl.when(s == 0)
    def _():
        _fill1d(zb_v, 2048, 0.0)
        def zc(j, carry):
            pltpu.sync_copy(zb_v, cnt_sh.at[pl.ds(j * 2048, 2048)])
            return carry
        lax.fori_loop(0, NP // 2048, zc, 0)

    plsc.subcore_barrier()

    @pl.when(c == 0)
    def _():
        pltpu.sync_copy(ei_hbm.at[0, s], idx2)

    @pl.when(c == 1)
    def _():
        pltpu.sync_copy(ei_hbm.at[1, s], idx2)

    def fire(j, carry):
        pltpu.async_copy(ones_v, cnt_sh.at[idx2.at[j]], sem, add=True)
        return carry
    lax.fori_loop(0, NB, fire, 0)

    def drain(j, carry):
        pltpu.make_async_copy(ones_v, cnt_sh.at[idx2.at[0]], sem).wait()
        return carry
    lax.fori_loop(0, NB, drain, 0)

    plsc.subcore_barrier()

    @pl.when(s == 0)
    def _():
        def dump(j, carry):
            pltpu.sync_copy(cnt_sh.at[pl.ds(j * 2048, 2048)], zb_v)
            pltpu.sync_copy(zb_v, deg_hbm.at[pl.ds(c * NP + j * 2048, 2048)])
            return carry
        lax.fori_loop(0, NP // 2048, dump, 0)


NSLOT = 4


@functools.partial(
    pl.kernel,
    out_type=(
        jax.ShapeDtypeStruct((NP, HALF), jnp.float32),
        jax.ShapeDtypeStruct((NP, HALF), jnp.float32),
    ),
    mesh=_mesh,
    scratch_types=(
        [pltpu.VMEM((CHS, KS), jnp.int32)] * 2 +
        [pltpu.VMEM((KS, HALF), jnp.float32)] * NSLOT +
        [pltpu.VMEM_SHARED((NP, HALF), jnp.float32)] +
        [pltpu.SemaphoreType.DMA] * (2 * NSLOT)
    ),
)
def _scat_kernel(ei_hbm, h0_hbm, h1_hbm, a0_hbm, a1_hbm,
                 sidx2, didx2, r0, r1, r2, r3, agg_sh,
                 g0, g1, g2, g3, s0, s1, s2, s3):
    c = lax.axis_index("c")
    s = lax.axis_index("s")
    start = s * RPT
    rows = (r0, r1, r2, r3)
    gsem = (g0, g1, g2, g3)
    ssem = (s0, s1, s2, s3)

    def zrow(j, carry):
        for k2 in range(HALF // LANES):
            r0[j, pl.ds(k2 * LANES, LANES)] = jnp.zeros((LANES,), jnp.float32)
        return carry
    lax.fori_loop(0, KS, zrow, 0)

    def zc(j, carry):
        pltpu.sync_copy(r0, agg_sh.at[pl.ds(start + j * KS, KS)])
        return carry
    lax.fori_loop(0, RPT // KS, zc, 0)

    plsc.subcore_barrier()

    def run(h_hbm):
        def gather(b, u):
            pltpu.async_copy(h_hbm.at[sidx2.at[b]], rows[u], gsem[u])

        def wait_g(u):
            pltpu.make_async_copy(h_hbm.at[pl.ds(0, KS)], rows[u],
                                  gsem[u]).wait()

        def scat(b, u):
            pltpu.async_copy(rows[u], agg_sh.at[didx2.at[b]], ssem[u],
                             add=True)

        def wait_s(u):
            pltpu.make_async_copy(rows[u], agg_sh.at[didx2.at[0]],
                                  ssem[u]).wait()

        def chunk(ch, carry):
            pltpu.sync_copy(ei_hbm.at[0, s, ch], sidx2)
            pltpu.sync_copy(ei_hbm.at[1, s, ch], didx2)
            for u in range(NSLOT):
                gather(u, u)

            def body(k, c2):
                for u in range(NSLOT):
                    wait_g(u)
                    scat(NSLOT * k + u, u)
                for u in range(NSLOT):
                    wait_s(u)
                    gather(NSLOT * k + u + NSLOT, u)
                return c2
            nf = (CHS - NSLOT) // NSLOT
            lax.fori_loop(0, nf, body, 0)

            wait_g(0)
            scat(NSLOT * nf, 0)
            wait_s(0)
            gather(CHS - 1, 0)
            for u in range(1, NSLOT):
                wait_g(u)
                scat(NSLOT * nf + u, u)
            wait_g(0)
            scat(CHS - 1, 0)
            for u in range(NSLOT):
                wait_s(u)
            return carry
        lax.fori_loop(0, NCH, chunk, 0)

    @pl.when(c == 0)
    def _():
        run(h0_hbm)

    @pl.when(c == 1)
    def _():
        run(h1_hbm)

    plsc.subcore_barrier()

    def dump(a_hbm):
        pltpu.sync_copy(agg_sh.at[pl.ds(start, RPT)],
                        a_hbm.at[pl.ds(start, RPT)])

    @pl.when(c == 0)
    def _():
        dump(a0_hbm)

    @pl.when(c == 1)
    def _():
        dump(a1_hbm)


def _scale_body(deg_ref, feat_ref, h0_ref, h1_ref):
    scale = lax.rsqrt(jnp.maximum(deg_ref[...], 1.0))
    hb = feat_ref[...] * scale
    h0_ref[...] = hb[:, :HALF]
    h1_ref[...] = hb[:, HALF:]


_scale_call = pl.pallas_call(
    _scale_body,
    grid=(GRID,),
    in_specs=[
        pl.BlockSpec((BR, 1), lambda i: (i, 0)),
        pl.BlockSpec((BR, D), lambda i: (i, 0)),
    ],
    out_specs=[
        pl.BlockSpec((BR, HALF), lambda i: (i, 0)),
        pl.BlockSpec((BR, HALF), lambda i: (i, 0)),
    ],
    out_shape=[jax.ShapeDtypeStruct((NP, HALF), jnp.float32)] * 2,
)


def _mm_bn_body(deg_ref, w_ref, b_ref, gamma_ref, beta_ref, a0_ref, a1_ref,
                o_ref, hbuf_ref, acc_ref):
    p = pl.program_id(0)
    i = pl.program_id(1)

    @pl.when(p == 0)
    def _():
        sin = lax.rsqrt(jnp.maximum(deg_ref[...], 1.0))
        a = jnp.concatenate([a0_ref[...], a1_ref[...]], axis=1) * sin
        hb = (jnp.dot(a, w_ref[...], preferred_element_type=jnp.float32)
              + b_ref[...])
        hbuf_ref[pl.ds(i * BR, BR), :] = hb
        o_ref[...] = hb

        rowid = i * BR + lax.broadcasted_iota(jnp.int32, (BR, 1), 0)
        hm = jnp.where(rowid < N, hb, 0.0)

        @pl.when(i == 0)
        def _():
            acc_ref[...] = jnp.zeros_like(acc_ref)

        acc_ref[...] += jnp.stack([jnp.sum(hm, axis=0),
                                   jnp.sum(hm * hm, axis=0)])

    @pl.when(p == 1)
    def _():
        mean = acc_ref[0, :] * (1.0 / N)
        ex2 = acc_ref[1, :] * (1.0 / N)
        var = ex2 - mean * mean
        inv = lax.rsqrt(var + BN_EPS)
        hb = hbuf_ref[pl.ds(i * BR, BR), :]
        o_ref[...] = ((hb - mean[None, :]) * (inv * gamma_ref[0, :])[None, :]
                      + beta_ref[0, :][None, :])


_mm_bn_call = pl.pallas_call(
    _mm_bn_body,
    grid=(2, GRID),
    in_specs=[
        pl.BlockSpec((BR, 1), lambda p, i: (NP // BR + i, 0)),
        pl.BlockSpec((D, D), lambda p, i: (0, 0)),
        pl.BlockSpec((1, D), lambda p, i: (0, 0)),
        pl.BlockSpec((1, D), lambda p, i: (0, 0)),
        pl.BlockSpec((1, D), lambda p, i: (0, 0)),
        pl.BlockSpec((BR, HALF), lambda p, i: (i * (1 - p), 0)),
        pl.BlockSpec((BR, HALF), lambda p, i: (i * (1 - p), 0)),
    ],
    out_specs=pl.BlockSpec((BR, D), lambda p, i: (i * p, 0)),
    out_shape=jax.ShapeDtypeStruct((N, D), jnp.float32),
    scratch_shapes=[
        pltpu.VMEM((NP, D), jnp.float32),
        pltpu.VMEM((2, D), jnp.float32),
    ],
)


def kernel(feat, edge_index, W, b, gamma, beta):
    ei = edge_index.astype(jnp.int32)
    deg = _deg_kernel(ei.reshape(2, NS, NB, K)).reshape(2 * NP, 1)
    h0, h1 = _scale_call(deg, feat)
    a0, a1 = _scat_kernel(ei.reshape(2, NS, NCH, CHS, KS), h0, h1)
    return _mm_bn_call(deg, W, b.reshape(1, D), gamma.reshape(1, D),
                       beta.reshape(1, D), a0, a1)
